# baseline (speedup 1.0000x reference)
import functools

import jax
import jax.numpy as jnp
from jax.experimental import pallas as pl

TOPK_K = 256
ROWS = 128
COLS = 32768
BLOCK_ROWS = 64
PHASE_A_ITERS = 8
PHASE_B_CAP = 16


def _topk_mask_body(x_ref, o_ref):
    kf = jnp.float32(TOPK_K)
    logk = jnp.log(jnp.float32(TOPK_K))
    x = x_ref[...]
    rows = x.shape[0]

    fold = x
    while fold.shape[1] > 4096:
        h = fold.shape[1] // 2
        fold = jnp.maximum(fold[:, :h], fold[:, h:])
    m3 = fold
    while fold.shape[1] > 256:
        h = fold.shape[1] // 2
        fold = jnp.maximum(fold[:, :h], fold[:, h:])
    mf = fold
    while mf.shape[1] > 1:
        h = mf.shape[1] // 2
        mf = jnp.minimum(mf[:, :h], mf[:, h:])
    lb = mf
    while fold.shape[1] > 1:
        h = fold.shape[1] // 2
        fold = jnp.maximum(fold[:, :h], fold[:, h:])
    ub = fold

    def interp_mid(lo, clo, hi, chi):
        num = jnp.log(clo) - logk
        den = jnp.log(clo) - jnp.log(jnp.maximum(chi, jnp.float32(0.5)))
        mid = lo + (num / den) * (hi - lo)
        return jnp.where((mid > lo) & (mid < hi), mid, jnp.float32(0.5) * (lo + hi))

    lo, hi = lb, ub
    clo = jnp.full((rows, 1), jnp.float32(4096.0))
    chi = jnp.full((rows, 1), jnp.float32(1.0))
    for _ in range(PHASE_A_ITERS):
        mid = interp_mid(lo, clo, hi, chi)
        c = jnp.sum(
            jnp.where(m3 >= mid, jnp.float32(1.0), jnp.float32(0.0)),
            axis=1,
            keepdims=True,
        )
        gek = c >= kf
        lo = jnp.where(gek, mid, lo)
        clo = jnp.where(gek, c, clo)
        hi = jnp.where(gek, hi, mid)
        chi = jnp.where(gek, chi, c)

    ca = jnp.sum(
        jnp.where(x >= lo, jnp.float32(1.0), jnp.float32(0.0)),
        axis=1,
        keepdims=True,
    )
    cb = jnp.sum(
        jnp.where(x >= hi, jnp.float32(1.0), jnp.float32(0.0)),
        axis=1,
        keepdims=True,
    )
    one = jnp.float32(1.0)
    hit_a = (ca >= kf) & (ca <= kf + one)
    hit_b = (cb >= kf) & (cb <= kf + one)
    done = jnp.where(hit_a | hit_b, jnp.int32(1), jnp.int32(0))
    tsel = jnp.where(hit_a, lo, hi)
    dsel = jnp.where(hit_a, ca - kf, cb - kf)
    clo = ca
    okhi = cb < kf
    hi = jnp.where(okhi, hi, ub)
    chi = jnp.where(okhi, cb, jnp.float32(1.0))

    def cond(state):
        lo, clo, hi, chi, tsel, dsel, done, it = state
        return jnp.logical_and(it < PHASE_B_CAP, jnp.any(done == jnp.int32(0)))

    def body(state):
        lo, clo, hi, chi, tsel, dsel, done, it = state
        mid = interp_mid(lo, clo, hi, chi)
        c = jnp.sum(
            jnp.where(x >= mid, jnp.float32(1.0), jnp.float32(0.0)),
            axis=1,
            keepdims=True,
        )
        active = done == jnp.int32(0)
        hit = active & (c >= kf) & (c <= kf + one)
        tsel = jnp.where(hit, mid, tsel)
        dsel = jnp.where(hit, c - kf, dsel)
        done = jnp.where(hit, jnp.int32(1), done)
        upd_lo = active & (c > kf)
        upd_hi = active & (c < kf)
        lo = jnp.where(upd_lo, mid, lo)
        clo = jnp.where(upd_lo, c, clo)
        hi = jnp.where(upd_hi, mid, hi)
        chi = jnp.where(upd_hi, c, chi)
        return lo, clo, hi, chi, tsel, dsel, done, it + jnp.int32(1)

    state = (lo, clo, hi, chi, tsel, dsel, done, jnp.int32(0))
    lo, clo, hi, chi, tsel, dsel, done, _ = jax.lax.while_loop(cond, body, state)

    m1 = jnp.min(
        jnp.where(x >= tsel, x, jnp.float32(jnp.inf)), axis=1, keepdims=True
    )
    ceq = jnp.sum(
        jnp.where(x == m1, jnp.float32(1.0), jnp.float32(0.0)),
        axis=1,
        keepdims=True,
    )
    row_ok = (done != jnp.int32(0)) & ((dsel == jnp.float32(0.0)) | (ceq == one))
    fast_ok = jnp.all(row_ok)

    @pl.when(fast_ok)
    def _():
        keep = (x >= tsel) & ((dsel == jnp.float32(0.0)) | (x != m1))
        o_ref[...] = x * jnp.where(keep, jnp.float32(1.0), jnp.float32(0.0))

    @pl.when(jnp.logical_not(fast_ok))
    def _():
        SIGNFLIP = jnp.int32(-(2**31))
        i = jax.lax.bitcast_convert_type(x, jnp.int32)
        ikey = i ^ ((i >> jnp.int32(31)) & jnp.int32(0x7FFFFFFF))

        t = jnp.zeros((rows, 1), dtype=jnp.int32)
        for b in range(31, -1, -1):
            bit = jnp.int32(-(2**31)) if b == 31 else jnp.int32(1 << b)
            cand = t | bit
            cnt = jnp.sum(
                jnp.where(
                    ikey >= (cand ^ SIGNFLIP), jnp.float32(1.0), jnp.float32(0.0)
                ),
                axis=1,
                keepdims=True,
            )
            t = jnp.where(cnt >= kf, cand, t)
        itf = t ^ SIGNFLIP

        gt = ikey > itf
        cnt_gt = jnp.sum(
            jnp.where(gt, jnp.float32(1.0), jnp.float32(0.0)),
            axis=1,
            keepdims=True,
        )
        need_eq = kf - cnt_gt
        eq = ikey == itf
        idx = jax.lax.broadcasted_iota(jnp.int32, x.shape, 1)
        m = jnp.zeros((rows, 1), dtype=jnp.int32)
        for b in range(14, -1, -1):
            cand = m | jnp.int32(1 << b)
            cnt = jnp.sum(
                jnp.where(eq & (idx < cand), jnp.float32(1.0), jnp.float32(0.0)),
                axis=1,
                keepdims=True,
            )
            m = jnp.where(cnt < need_eq, cand, m)
        keep = gt | (eq & (idx <= m))
        o_ref[...] = x * jnp.where(keep, jnp.float32(1.0), jnp.float32(0.0))


@functools.partial(jax.jit)
def kernel(input):
    return pl.pallas_call(
        _topk_mask_body,
        grid=(ROWS // BLOCK_ROWS,),
        in_specs=[pl.BlockSpec((BLOCK_ROWS, COLS), lambda i: (i, 0))],
        out_specs=pl.BlockSpec((BLOCK_ROWS, COLS), lambda i: (i, 0)),
        out_shape=jax.ShapeDtypeStruct((ROWS, COLS), jnp.float32),
    )(input)


# speedup vs baseline: 1.0452x; 1.0452x over previous
import functools

import jax
import jax.numpy as jnp
from jax.experimental import pallas as pl

TOPK_K = 256
ROWS = 128
COLS = 32768
BLOCK_ROWS = 64
PHASE_A_ITERS = 8
PHASE_B_CAP = 16


def _topk_mask_body(x_ref, o_ref):
    kf = jnp.float32(TOPK_K)
    logk = jnp.log(jnp.float32(TOPK_K))
    x = x_ref[...]
    rows = x.shape[0]

    mn = jnp.min(x, axis=1, keepdims=True)
    mx = jnp.max(x, axis=1, keepdims=True)
    s1 = jnp.sum(x, axis=1, keepdims=True)
    s2 = jnp.sum(x * x, axis=1, keepdims=True)
    inv_n = jnp.float32(1.0 / COLS)
    mu = s1 * inv_n
    sd = jnp.sqrt(jnp.maximum(s2 * inv_n - mu * mu, jnp.float32(0.0)))
    pa = mu + jnp.float32(2.25) * sd
    pb = mu + jnp.float32(2.6) * sd

    def interp_mid(lo, clo, hi, chi):
        num = jnp.log(clo) - logk
        den = jnp.log(clo) - jnp.log(jnp.maximum(chi, jnp.float32(0.5)))
        mid = lo + (num / den) * (hi - lo)
        return jnp.where((mid > lo) & (mid < hi), mid, jnp.float32(0.5) * (lo + hi))

    ca = jnp.sum(
        jnp.where(x >= pa, jnp.float32(1.0), jnp.float32(0.0)),
        axis=1,
        keepdims=True,
    )
    cb = jnp.sum(
        jnp.where(x >= pb, jnp.float32(1.0), jnp.float32(0.0)),
        axis=1,
        keepdims=True,
    )
    one = jnp.float32(1.0)
    hit_a = (ca >= kf) & (ca <= kf + one)
    hit_b = (cb >= kf) & (cb <= kf + one)
    done = jnp.where(hit_a | hit_b, jnp.int32(1), jnp.int32(0))
    tsel = jnp.where(hit_a, pa, pb)
    dsel = jnp.where(hit_a, ca - kf, cb - kf)
    oklo = ca >= kf
    lo = jnp.where(oklo, pa, mn)
    clo = jnp.where(oklo, ca, jnp.float32(float(COLS)))
    okhi = cb < kf
    hi = jnp.where(okhi, pb, mx)
    chi = jnp.where(okhi, cb, one)

    def cond(state):
        lo, clo, hi, chi, tsel, dsel, done, it = state
        return jnp.logical_and(it < PHASE_B_CAP, jnp.any(done == jnp.int32(0)))

    def body(state):
        lo, clo, hi, chi, tsel, dsel, done, it = state
        mid = interp_mid(lo, clo, hi, chi)
        c = jnp.sum(
            jnp.where(x >= mid, jnp.float32(1.0), jnp.float32(0.0)),
            axis=1,
            keepdims=True,
        )
        active = done == jnp.int32(0)
        hit = active & (c >= kf) & (c <= kf + one)
        tsel = jnp.where(hit, mid, tsel)
        dsel = jnp.where(hit, c - kf, dsel)
        done = jnp.where(hit, jnp.int32(1), done)
        upd_lo = active & (c > kf)
        upd_hi = active & (c < kf)
        lo = jnp.where(upd_lo, mid, lo)
        clo = jnp.where(upd_lo, c, clo)
        hi = jnp.where(upd_hi, mid, hi)
        chi = jnp.where(upd_hi, c, chi)
        return lo, clo, hi, chi, tsel, dsel, done, it + jnp.int32(1)

    state = (lo, clo, hi, chi, tsel, dsel, done, jnp.int32(0))
    lo, clo, hi, chi, tsel, dsel, done, _ = jax.lax.while_loop(cond, body, state)

    m1 = jnp.min(
        jnp.where(x >= tsel, x, jnp.float32(jnp.inf)), axis=1, keepdims=True
    )
    ceq = jnp.sum(
        jnp.where(x == m1, jnp.float32(1.0), jnp.float32(0.0)),
        axis=1,
        keepdims=True,
    )
    row_ok = (done != jnp.int32(0)) & ((dsel == jnp.float32(0.0)) | (ceq == one))
    fast_ok = jnp.all(row_ok)

    @pl.when(fast_ok)
    def _():
        keep = (x >= tsel) & ((dsel == jnp.float32(0.0)) | (x != m1))
        o_ref[...] = x * jnp.where(keep, jnp.float32(1.0), jnp.float32(0.0))

    @pl.when(jnp.logical_not(fast_ok))
    def _():
        SIGNFLIP = jnp.int32(-(2**31))
        i = jax.lax.bitcast_convert_type(x, jnp.int32)
        ikey = i ^ ((i >> jnp.int32(31)) & jnp.int32(0x7FFFFFFF))

        t = jnp.zeros((rows, 1), dtype=jnp.int32)
        for b in range(31, -1, -1):
            bit = jnp.int32(-(2**31)) if b == 31 else jnp.int32(1 << b)
            cand = t | bit
            cnt = jnp.sum(
                jnp.where(
                    ikey >= (cand ^ SIGNFLIP), jnp.float32(1.0), jnp.float32(0.0)
                ),
                axis=1,
                keepdims=True,
            )
            t = jnp.where(cnt >= kf, cand, t)
        itf = t ^ SIGNFLIP

        gt = ikey > itf
        cnt_gt = jnp.sum(
            jnp.where(gt, jnp.float32(1.0), jnp.float32(0.0)),
            axis=1,
            keepdims=True,
        )
        need_eq = kf - cnt_gt
        eq = ikey == itf
        idx = jax.lax.broadcasted_iota(jnp.int32, x.shape, 1)
        m = jnp.zeros((rows, 1), dtype=jnp.int32)
        for b in range(14, -1, -1):
            cand = m | jnp.int32(1 << b)
            cnt = jnp.sum(
                jnp.where(eq & (idx < cand), jnp.float32(1.0), jnp.float32(0.0)),
                axis=1,
                keepdims=True,
            )
            m = jnp.where(cnt < need_eq, cand, m)
        keep = gt | (eq & (idx <= m))
        o_ref[...] = x * jnp.where(keep, jnp.float32(1.0), jnp.float32(0.0))


@functools.partial(jax.jit)
def kernel(input):
    return pl.pallas_call(
        _topk_mask_body,
        grid=(ROWS // BLOCK_ROWS,),
        in_specs=[pl.BlockSpec((BLOCK_ROWS, COLS), lambda i: (i, 0))],
        out_specs=pl.BlockSpec((BLOCK_ROWS, COLS), lambda i: (i, 0)),
        out_shape=jax.ShapeDtypeStruct((ROWS, COLS), jnp.float32),
    )(input)


# speedup vs baseline: 1.0905x; 1.0433x over previous
import functools

import jax
import jax.numpy as jnp
from jax.experimental import pallas as pl

TOPK_K = 256
ROWS = 128
COLS = 32768
BLOCK_ROWS = 64
PHASE_A_ITERS = 8
PHASE_B_CAP = 16


def _topk_mask_body(x_ref, o_ref):
    kf = jnp.float32(TOPK_K)
    logk = jnp.log(jnp.float32(TOPK_K))
    x = x_ref[...]
    rows = x.shape[0]

    s1 = jnp.sum(x, axis=1, keepdims=True)
    s2 = jnp.sum(x * x, axis=1, keepdims=True)
    inv_n = jnp.float32(1.0 / COLS)
    mu = s1 * inv_n
    sd = jnp.sqrt(jnp.maximum(s2 * inv_n - mu * mu, jnp.float32(0.0)))
    pa = mu + jnp.float32(2.25) * sd
    pb = mu + jnp.float32(2.6) * sd

    def interp_mid(lo, clo, hi, chi):
        num = jnp.log(clo) - logk
        den = jnp.log(clo) - jnp.log(jnp.maximum(chi, jnp.float32(0.5)))
        mid = lo + (num / den) * (hi - lo)
        mid = jnp.where((mid > lo) & (mid < hi), mid, jnp.float32(0.5) * (lo + hi))
        return jnp.where(jnp.isfinite(mid), mid, mu)

    ca = jnp.sum(
        jnp.where(x >= pa, jnp.float32(1.0), jnp.float32(0.0)),
        axis=1,
        keepdims=True,
    )
    cb = jnp.sum(
        jnp.where(x >= pb, jnp.float32(1.0), jnp.float32(0.0)),
        axis=1,
        keepdims=True,
    )
    one = jnp.float32(1.0)
    hit_a = (ca >= kf) & (ca <= kf + one)
    hit_b = (cb >= kf) & (cb <= kf + one)
    done = jnp.where(hit_a | hit_b, jnp.int32(1), jnp.int32(0))
    tsel = jnp.where(hit_a, pa, pb)
    dsel = jnp.where(hit_a, ca - kf, cb - kf)
    oklo = ca >= kf
    lo = jnp.where(oklo, pa, -jnp.float32(jnp.inf))
    clo = jnp.where(oklo, ca, jnp.float32(float(COLS)))
    okhi = cb < kf
    hi = jnp.where(okhi, pb, jnp.float32(jnp.inf))
    chi = jnp.where(okhi, cb, one)

    def cond(state):
        lo, clo, hi, chi, tsel, dsel, done, it = state
        return jnp.logical_and(it < PHASE_B_CAP, jnp.any(done == jnp.int32(0)))

    def body(state):
        lo, clo, hi, chi, tsel, dsel, done, it = state
        mid = interp_mid(lo, clo, hi, chi)
        c = jnp.sum(
            jnp.where(x >= mid, jnp.float32(1.0), jnp.float32(0.0)),
            axis=1,
            keepdims=True,
        )
        active = done == jnp.int32(0)
        hit = active & (c >= kf) & (c <= kf + one)
        tsel = jnp.where(hit, mid, tsel)
        dsel = jnp.where(hit, c - kf, dsel)
        done = jnp.where(hit, jnp.int32(1), done)
        upd_lo = active & (c > kf)
        upd_hi = active & (c < kf)
        lo = jnp.where(upd_lo, mid, lo)
        clo = jnp.where(upd_lo, c, clo)
        hi = jnp.where(upd_hi, mid, hi)
        chi = jnp.where(upd_hi, c, chi)
        return lo, clo, hi, chi, tsel, dsel, done, it + jnp.int32(1)

    state = (lo, clo, hi, chi, tsel, dsel, done, jnp.int32(0))
    lo, clo, hi, chi, tsel, dsel, done, _ = jax.lax.while_loop(cond, body, state)

    m1 = jnp.min(
        jnp.where(x >= tsel, x, jnp.float32(jnp.inf)), axis=1, keepdims=True
    )
    ceq = jnp.sum(
        jnp.where(x == m1, jnp.float32(1.0), jnp.float32(0.0)),
        axis=1,
        keepdims=True,
    )
    drop = dsel != jnp.float32(0.0)
    row_ok = (done != jnp.int32(0)) & ((~drop) | (ceq == one))
    fast_ok = jnp.all(row_ok)

    @pl.when(fast_ok)
    def _():
        keep = (x >= tsel) & ((~drop) | (x != m1))
        o_ref[...] = x * jnp.where(keep, jnp.float32(1.0), jnp.float32(0.0))

    @pl.when(jnp.logical_not(fast_ok))
    def _():
        SIGNFLIP = jnp.int32(-(2**31))
        i = jax.lax.bitcast_convert_type(x, jnp.int32)
        ikey = i ^ ((i >> jnp.int32(31)) & jnp.int32(0x7FFFFFFF))

        t = jnp.zeros((rows, 1), dtype=jnp.int32)
        for b in range(31, -1, -1):
            bit = jnp.int32(-(2**31)) if b == 31 else jnp.int32(1 << b)
            cand = t | bit
            cnt = jnp.sum(
                jnp.where(
                    ikey >= (cand ^ SIGNFLIP), jnp.float32(1.0), jnp.float32(0.0)
                ),
                axis=1,
                keepdims=True,
            )
            t = jnp.where(cnt >= kf, cand, t)
        itf = t ^ SIGNFLIP

        gt = ikey > itf
        cnt_gt = jnp.sum(
            jnp.where(gt, jnp.float32(1.0), jnp.float32(0.0)),
            axis=1,
            keepdims=True,
        )
        need_eq = kf - cnt_gt
        eq = ikey == itf
        idx = jax.lax.broadcasted_iota(jnp.int32, x.shape, 1)
        m = jnp.zeros((rows, 1), dtype=jnp.int32)
        for b in range(14, -1, -1):
            cand = m | jnp.int32(1 << b)
            cnt = jnp.sum(
                jnp.where(eq & (idx < cand), jnp.float32(1.0), jnp.float32(0.0)),
                axis=1,
                keepdims=True,
            )
            m = jnp.where(cnt < need_eq, cand, m)
        keep = gt | (eq & (idx <= m))
        o_ref[...] = x * jnp.where(keep, jnp.float32(1.0), jnp.float32(0.0))


@functools.partial(jax.jit)
def kernel(input):
    return pl.pallas_call(
        _topk_mask_body,
        grid=(ROWS // BLOCK_ROWS,),
        in_specs=[pl.BlockSpec((BLOCK_ROWS, COLS), lambda i: (i, 0))],
        out_specs=pl.BlockSpec((BLOCK_ROWS, COLS), lambda i: (i, 0)),
        out_shape=jax.ShapeDtypeStruct((ROWS, COLS), jnp.float32),
    )(input)
